# baseline (speedup 1.0000x reference)
import functools

import jax
import jax.numpy as jnp
from jax import lax
from jax.experimental import pallas as pl
from jax.experimental.pallas import tpu as pltpu
from jax.experimental.pallas import tpu_sc as plsc

_N = 10000
_E = 320000
_D = 128

_NC = 2
_NS = 16
_NW = _NC * _NS
_EPW_DEG = _E // _NW
_CH = 80
_EPW = _E // _NW
_NCH = _EPW // _CH
_NACC = _N
_NSLOT = 4
_GD = 3
_SD = 1
assert _SD <= _NSLOT - _GD and _EPW % _CH == 0 and _CH % 8 == 0
_RPN = 624
_RPN_LAST = _N - 15 * _RPN


_MESH = plsc.VectorSubcoreMesh(
    core_axis_name="c", subcore_axis_name="s", num_cores=_NC, num_subcores=_NS
)


@functools.partial(
    pl.kernel,
    out_type=jax.ShapeDtypeStruct((_NW, _N), jnp.float32),
    mesh=_MESH,
    scratch_types=[
        pltpu.VMEM((_EPW_DEG,), jnp.int32),
        pltpu.VMEM((_N,), jnp.float32),
    ],
    compiler_params=pltpu.CompilerParams(needs_layout_passes=False),
)
def _deg_kernel(dst_hbm, out_hbm, idx_v, deg_v):
    c = lax.axis_index("c")
    s = lax.axis_index("s")
    wid = s * _NC + c

    def zero_body(i, carry):
        deg_v[pl.ds(i * 16, 16)] = jnp.zeros((16,), jnp.float32)
        return carry

    lax.fori_loop(0, _N // 16, zero_body, 0)

    pltpu.sync_copy(dst_hbm.at[pl.ds(wid * _EPW_DEG, _EPW_DEG)], idx_v)
    ones = jnp.ones((16,), jnp.float32)

    def body(i, carry):
        idx16 = idx_v[pl.ds(i * 16, 16)]
        plsc.addupdate_scatter(deg_v, [idx16], ones)
        return carry

    lax.fori_loop(0, _EPW_DEG // 16, body, 0)
    pltpu.sync_copy(deg_v, out_hbm.at[wid])


@functools.partial(
    pl.kernel,
    out_type=jax.ShapeDtypeStruct((_NC, _N, _D), jnp.float32),
    mesh=_MESH,
    scratch_types=[
        [pltpu.VMEM((_CH,), jnp.int32)] * _NSLOT,
        [pltpu.VMEM((_CH,), jnp.int32)] * _NSLOT,
        [pltpu.VMEM((_CH, _D), jnp.float32)] * _NSLOT,
        pltpu.VMEM_SHARED((_NACC, _D), jnp.float32),
        [pltpu.SemaphoreType.DMA] * _NSLOT,
        [pltpu.SemaphoreType.DMA] * _NSLOT,
        [pltpu.SemaphoreType.DMA] * _NSLOT,
        [pltpu.SemaphoreType.DMA] * _NSLOT,
    ],
    compiler_params=pltpu.CompilerParams(needs_layout_passes=False),
)
def _agg_kernel(y_hbm, src_hbm, dst_hbm, zeros_hbm, out_hbm,
                isv, idv, bufs, acc_sp, sem_is, sem_id, sem_g, sem_s):
    c = lax.axis_index("c")
    s = lax.axis_index("s")
    wid = s * _NC + c
    ebase = wid * _EPW

    def fetch(hbm, j, dst, sem):
        eoff = pl.multiple_of(ebase + j * _CH, 8)
        pltpu.async_copy(hbm.at[pl.ds(eoff, _CH)], dst, sem)

    def drain(src, dst, sem):
        pltpu.make_async_copy(src, dst, sem).wait()

    for k in range(_NSLOT):
        fetch(src_hbm, k, isv[k], sem_is[k])
        fetch(dst_hbm, k, idv[k], sem_id[k])
    for k in range(_GD):
        drain(src_hbm.at[pl.ds(0, _CH)], isv[k], sem_is[k])
        pltpu.async_copy(y_hbm.at[isv[k]], bufs[k], sem_g[k])

    off = pl.multiple_of(s * _RPN, 8)

    def _init_from(src):
        @pl.when(s < _NS - 1)
        def _():
            pltpu.sync_copy(src.at[pl.ds(off, _RPN)],
                            acc_sp.at[pl.ds(off, _RPN)])

        @pl.when(s == _NS - 1)
        def _():
            pltpu.sync_copy(src.at[pl.ds(15 * _RPN, _RPN_LAST)],
                            acc_sp.at[pl.ds(15 * _RPN, _RPN_LAST)])

    @pl.when(c == 0)
    def _():
        _init_from(y_hbm)

    @pl.when(c == 1)
    def _():
        _init_from(zeros_hbm)

    plsc.subcore_barrier()

    def step(t, sl):
        slD = (sl - _SD) % _NSLOT
        slG = (sl + _GD) % _NSLOT

        @pl.when(t >= _SD)
        def _():
            drain(bufs[slD], acc_sp.at[idv[slD]], sem_s[slD])

        @pl.when(jnp.logical_and(t >= _SD, t - _SD + _NSLOT < _NCH))
        def _():
            fetch(dst_hbm, t - _SD + _NSLOT, idv[slD], sem_id[slD])

        @pl.when(t + _GD < _NCH)
        def _():
            drain(src_hbm.at[pl.ds(0, _CH)], isv[slG], sem_is[slG])
            pltpu.async_copy(y_hbm.at[isv[slG]], bufs[slG], sem_g[slG])

        drain(y_hbm.at[isv[sl]], bufs[sl], sem_g[sl])
        drain(dst_hbm.at[pl.ds(0, _CH)], idv[sl], sem_id[sl])
        pltpu.async_copy(bufs[sl], acc_sp.at[idv[sl]], sem_s[sl], add=True)

        @pl.when(t + _NSLOT < _NCH)
        def _():
            fetch(src_hbm, t + _NSLOT, isv[sl], sem_is[sl])

    def round_body(q, carry):
        t = _NSLOT * q
        for k in range(_NSLOT):
            step(t + k, k)
        return carry

    lax.fori_loop(0, _NCH // _NSLOT, round_body, 0)
    for r in range(_NCH % _NSLOT):
        step((_NCH // _NSLOT) * _NSLOT + r, r)

    for j in range(_NCH - _SD, _NCH):
        sl = j % _NSLOT
        drain(bufs[sl], acc_sp.at[idv[sl]], sem_s[sl])

    plsc.subcore_barrier()

    @pl.when(s < _NS - 1)
    def _():
        pltpu.sync_copy(acc_sp.at[pl.ds(off, _RPN)],
                        out_hbm.at[c, pl.ds(off, _RPN)])

    @pl.when(s == _NS - 1)
    def _():
        pltpu.sync_copy(acc_sp.at[pl.ds(15 * _RPN, _RPN_LAST)],
                        out_hbm.at[c, pl.ds(15 * _RPN, _RPN_LAST)])


_BR = 400
_GRID = _N // _BR


def _tc0_body(deg_ref, dinv_ref):
    deg = jnp.sum(deg_ref[...], axis=0) + 1.0
    dinv_ref[...] = lax.rsqrt(deg)[:, None]


def _tc1_body(dinv_ref, x_ref, w_ref, y_ref):
    xw = jnp.dot(x_ref[...], w_ref[...], preferred_element_type=jnp.float32)
    y_ref[...] = xw * dinv_ref[...]


def _tc2_body(dinv_ref, a_ref, b1_ref, w2_ref, y2_ref):
    dinv = dinv_ref[...]
    h = jnp.maximum((a_ref[0] + a_ref[1]) * dinv + b1_ref[...], 0.0)
    y2_ref[...] = jnp.dot(h, w2_ref[...], preferred_element_type=jnp.float32) * dinv


def _tc3_body(dinv_ref, a_ref, b2_ref, o_ref):
    o_ref[...] = (a_ref[0] + a_ref[1]) * dinv_ref[...] + b2_ref[...]


_f32 = jnp.float32
_dinv_spec = pl.BlockSpec((_BR, 1), lambda i: (i, 0))
_row_spec = pl.BlockSpec((_BR, _D), lambda i: (i, 0))
_acc_spec = pl.BlockSpec((_NC, _BR, _D), lambda i: (0, i, 0))
_w_spec = pl.BlockSpec((_D, _D), lambda i: (0, 0))
_b_spec = pl.BlockSpec((_D,), lambda i: (0,))

_tc0 = pl.pallas_call(_tc0_body, out_shape=jax.ShapeDtypeStruct((_N, 1), _f32))
_tc1 = pl.pallas_call(
    _tc1_body,
    grid=(_GRID,),
    in_specs=[_dinv_spec, _row_spec, _w_spec],
    out_specs=_row_spec,
    out_shape=jax.ShapeDtypeStruct((_N, _D), _f32),
)
_tc2 = pl.pallas_call(
    _tc2_body,
    grid=(_GRID,),
    in_specs=[_dinv_spec, _acc_spec, _b_spec, _w_spec],
    out_specs=_row_spec,
    out_shape=jax.ShapeDtypeStruct((_N, _D), _f32),
)
_tc3 = pl.pallas_call(
    _tc3_body,
    grid=(_GRID,),
    in_specs=[_dinv_spec, _acc_spec, _b_spec],
    out_specs=_row_spec,
    out_shape=jax.ShapeDtypeStruct((_N, _D), _f32),
)


def kernel(x, adj, W1, b1, W2, b2):
    src = adj[0].astype(jnp.int32)
    dst = adj[1].astype(jnp.int32)
    srcp, dstp = src, dst
    zeros = jnp.zeros((_N, _D), _f32)

    deg = _deg_kernel(dst)
    dinv = _tc0(deg)
    y1 = _tc1(dinv, x, W1)
    a1 = _agg_kernel(y1, srcp, dstp, zeros)
    y2 = _tc2(dinv, a1, b1, W2)
    a2 = _agg_kernel(y2, srcp, dstp, zeros)
    out = _tc3(dinv, a2, b2)
    return out


# speedup vs baseline: 1.1580x; 1.1580x over previous
import functools

import jax
import jax.numpy as jnp
from jax import lax
from jax.experimental import pallas as pl
from jax.experimental.pallas import tpu as pltpu
from jax.experimental.pallas import tpu_sc as plsc

_N = 10000
_E = 320000
_D = 128

_NC = 2
_NS = 16
_NW = _NC * _NS
_EPW_DEG = _E // _NW
_CH = 80
_EPW = _E // _NW
_NCH = _EPW // _CH
_NACC = _N
_NSLOT = 4
_GD = 3
_SD = 1
assert _SD <= _NSLOT - _GD and _EPW % _CH == 0 and _CH % 8 == 0
_RPN = 624
_RPN_LAST = _N - 15 * _RPN


_MESH = plsc.VectorSubcoreMesh(
    core_axis_name="c", subcore_axis_name="s", num_cores=_NC, num_subcores=_NS
)


@functools.partial(
    pl.kernel,
    out_type=jax.ShapeDtypeStruct((_NW, _N), jnp.float32),
    mesh=_MESH,
    scratch_types=[
        pltpu.VMEM((_EPW_DEG,), jnp.int32),
        pltpu.VMEM((_N,), jnp.float32),
    ],
    compiler_params=pltpu.CompilerParams(needs_layout_passes=False),
)
def _deg_kernel(dst_hbm, out_hbm, idx_v, deg_v):
    c = lax.axis_index("c")
    s = lax.axis_index("s")
    wid = s * _NC + c

    def zero_body(i, carry):
        deg_v[pl.ds(i * 16, 16)] = jnp.zeros((16,), jnp.float32)
        return carry

    lax.fori_loop(0, _N // 16, zero_body, 0)

    pltpu.sync_copy(dst_hbm.at[pl.ds(wid * _EPW_DEG, _EPW_DEG)], idx_v)
    ones = jnp.ones((16,), jnp.float32)

    def body(i, carry):
        idx16 = idx_v[pl.ds(i * 16, 16)]
        plsc.addupdate_scatter(deg_v, [idx16], ones)
        return carry

    lax.fori_loop(0, _EPW_DEG // 16, body, 0)
    pltpu.sync_copy(deg_v, out_hbm.at[wid])


@functools.partial(
    pl.kernel,
    out_type=jax.ShapeDtypeStruct((_NC, _N, _D), jnp.float32),
    mesh=_MESH,
    scratch_types=[
        [pltpu.VMEM((_CH,), jnp.int32)] * _NSLOT,
        [pltpu.VMEM((_CH,), jnp.int32)] * _NSLOT,
        [pltpu.VMEM((_CH, _D), jnp.float32)] * _NSLOT,
        pltpu.VMEM_SHARED((_NACC, _D), jnp.float32),
        [pltpu.SemaphoreType.DMA] * _NSLOT,
        [pltpu.SemaphoreType.DMA] * _NSLOT,
        [pltpu.SemaphoreType.DMA] * _NSLOT,
        [pltpu.SemaphoreType.DMA] * _NSLOT,
    ],
    compiler_params=pltpu.CompilerParams(needs_layout_passes=False),
)
def _agg_kernel(y_hbm, src_hbm, dst_hbm, zeros_hbm, out_hbm,
                isv, idv, bufs, acc_sp, sem_is, sem_id, sem_g, sem_s):
    c = lax.axis_index("c")
    s = lax.axis_index("s")
    wid = s * _NC + c
    ebase = wid * _EPW

    def fetch(hbm, j, dst, sem):
        eoff = pl.multiple_of(ebase + j * _CH, 8)
        pltpu.async_copy(hbm.at[pl.ds(eoff, _CH)], dst, sem)

    def drain(src, dst, sem):
        pltpu.make_async_copy(src, dst, sem).wait()

    for k in range(_NSLOT):
        fetch(src_hbm, k, isv[k], sem_is[k])
        fetch(dst_hbm, k, idv[k], sem_id[k])
    for k in range(_GD):
        drain(src_hbm.at[pl.ds(0, _CH)], isv[k], sem_is[k])
        pltpu.async_copy(y_hbm.at[isv[k]], bufs[k], sem_g[k])

    off = pl.multiple_of(s * _RPN, 8)

    def _init_from(src):
        @pl.when(s < _NS - 1)
        def _():
            pltpu.sync_copy(src.at[pl.ds(off, _RPN)],
                            acc_sp.at[pl.ds(off, _RPN)])

        @pl.when(s == _NS - 1)
        def _():
            pltpu.sync_copy(src.at[pl.ds(15 * _RPN, _RPN_LAST)],
                            acc_sp.at[pl.ds(15 * _RPN, _RPN_LAST)])

    @pl.when(c == 0)
    def _():
        _init_from(y_hbm)

    @pl.when(c == 1)
    def _():
        _init_from(zeros_hbm)

    plsc.subcore_barrier()

    def step(t, sl):
        slD = (sl - _SD) % _NSLOT
        slG = (sl + _GD) % _NSLOT

        @pl.when(t >= _SD)
        def _():
            drain(bufs[slD], acc_sp.at[idv[slD]], sem_s[slD])

        @pl.when(jnp.logical_and(t >= _SD, t - _SD + _NSLOT < _NCH))
        def _():
            fetch(dst_hbm, t - _SD + _NSLOT, idv[slD], sem_id[slD])

        @pl.when(t + _GD < _NCH)
        def _():
            drain(src_hbm.at[pl.ds(0, _CH)], isv[slG], sem_is[slG])
            pltpu.async_copy(y_hbm.at[isv[slG]], bufs[slG], sem_g[slG])

        drain(y_hbm.at[isv[sl]], bufs[sl], sem_g[sl])
        drain(dst_hbm.at[pl.ds(0, _CH)], idv[sl], sem_id[sl])
        pltpu.async_copy(bufs[sl], acc_sp.at[idv[sl]], sem_s[sl], add=True)

        @pl.when(t + _NSLOT < _NCH)
        def _():
            fetch(src_hbm, t + _NSLOT, isv[sl], sem_is[sl])

    def round_body(q, carry):
        t = _NSLOT * q
        for k in range(_NSLOT):
            step(t + k, k)
        return carry

    lax.fori_loop(0, _NCH // _NSLOT, round_body, 0)
    for r in range(_NCH % _NSLOT):
        step((_NCH // _NSLOT) * _NSLOT + r, r)

    for j in range(_NCH - _SD, _NCH):
        sl = j % _NSLOT
        drain(bufs[sl], acc_sp.at[idv[sl]], sem_s[sl])

    plsc.subcore_barrier()

    @pl.when(s < _NS - 1)
    def _():
        pltpu.sync_copy(acc_sp.at[pl.ds(off, _RPN)],
                        out_hbm.at[c, pl.ds(off, _RPN)])

    @pl.when(s == _NS - 1)
    def _():
        pltpu.sync_copy(acc_sp.at[pl.ds(15 * _RPN, _RPN_LAST)],
                        out_hbm.at[c, pl.ds(15 * _RPN, _RPN_LAST)])


_BR = 400
_GRID = _N // _BR


def _dinv_from_parts(deg_ref):
    deg = jnp.sum(deg_ref[...], axis=0) + 1.0
    return lax.rsqrt(deg)[:, None]


def _tc1_body(deg_ref, x_ref, w_ref, y_ref):
    xw = jnp.dot(x_ref[...], w_ref[...], preferred_element_type=jnp.float32)
    y_ref[...] = xw * _dinv_from_parts(deg_ref)


def _tc2_body(deg_ref, a_ref, b1_ref, w2_ref, y2_ref):
    dinv = _dinv_from_parts(deg_ref)
    h = jnp.maximum((a_ref[0] + a_ref[1]) * dinv + b1_ref[...], 0.0)
    y2_ref[...] = jnp.dot(h, w2_ref[...], preferred_element_type=jnp.float32) * dinv


def _tc3_body(deg_ref, a_ref, b2_ref, o_ref):
    o_ref[...] = (a_ref[0] + a_ref[1]) * _dinv_from_parts(deg_ref) + b2_ref[...]


_f32 = jnp.float32
_tc1 = pl.pallas_call(_tc1_body, out_shape=jax.ShapeDtypeStruct((_N, _D), _f32))
_tc2 = pl.pallas_call(_tc2_body, out_shape=jax.ShapeDtypeStruct((_N, _D), _f32))
_tc3 = pl.pallas_call(_tc3_body, out_shape=jax.ShapeDtypeStruct((_N, _D), _f32))


def kernel(x, adj, W1, b1, W2, b2):
    src = adj[0].astype(jnp.int32)
    dst = adj[1].astype(jnp.int32)
    srcp, dstp = src, dst
    zeros = jnp.zeros((_N, _D), _f32)

    deg = _deg_kernel(dst)
    y1 = _tc1(deg, x, W1)
    a1 = _agg_kernel(y1, srcp, dstp, zeros)
    y2 = _tc2(deg, a1, b1, W2)
    a2 = _agg_kernel(y2, srcp, dstp, zeros)
    out = _tc3(deg, a2, b2)
    return out
